# baseline (speedup 1.0000x reference)
import functools

import jax
import jax.numpy as jnp
from jax import lax
from jax.experimental import pallas as pl
from jax.experimental.pallas import tpu as pltpu
from jax.experimental.pallas import tpu_sc as plsc

B = 4096
F = 26
D = 32
DD = 13
NW = 32
CHUNK = 128
NIDX = B * F
IDX_PER_W = NIDX // NW
NCHUNK = IDX_PER_W // CHUNK
SEG_CH = 2
NSEG = NCHUNK // SEG_CH
SEG_ROWS = SEG_CH * CHUNK
SEG_LINES = SEG_ROWS // 4
NOUT = (F + DD) * D
NLINES = NIDX // 4


@functools.lru_cache(maxsize=None)
def _get_sc_gather():
    mesh = plsc.VectorSubcoreMesh(core_axis_name="c", subcore_axis_name="s")

    @functools.partial(
        pl.kernel,
        mesh=mesh,
        out_type=jax.ShapeDtypeStruct((NLINES, CHUNK), jnp.float32),
        scratch_types=[
            pltpu.VMEM((NCHUNK, CHUNK), jnp.int32),
            pltpu.VMEM((2, SEG_ROWS, CHUNK), jnp.float32),
            pltpu.VMEM((SEG_LINES, CHUNK), jnp.float32),
            pltpu.VMEM((IDX_PER_W,), jnp.int32),
            pltpu.SemaphoreType.DMA,
        ],
    )
    def _sc_gather(tabler_hbm, idxq_hbm, idxr_hbm, out_hbm,
                   idxq_v, quads_v, seg_v, roff_v, sem):
        wid = lax.axis_index("s") * 2 + lax.axis_index("c")
        pltpu.sync_copy(idxq_hbm.at[wid], idxq_v)
        pltpu.sync_copy(idxr_hbm.at[wid], roff_v)

        def fire(s, buf):
            return [
                pltpu.async_copy(
                    tabler_hbm.at[idxq_v.at[s * SEG_CH + c]],
                    quads_v.at[buf].at[pl.ds(c * CHUNK, CHUNK)],
                    sem,
                )
                for c in range(SEG_CH)
            ]

        pending = fire(0, 0)
        for s in range(NSEG):
            nxt = []
            if s + 1 < NSEG:
                nxt = fire(s + 1, (s + 1) % 2)
            for h in pending:
                h.wait()
            pending = nxt
            buf = s % 2
            qflat = quads_v.at[buf]

            def compact(gi, carry):
                base = gi * 16
                rv = roff_v[pl.ds(s * SEG_ROWS + base, 16)]
                for j in range(16):
                    i = base + j
                    r = rv[j]
                    line = gi * 4 + j // 4
                    lane0 = (j % 4) * D
                    for half in range(2):
                        seg_v.at[line][pl.ds(lane0 + half * 16, 16)] = (
                            qflat.at[i][pl.ds(r + half * 16, 16)]
                        )
                return carry

            lax.fori_loop(0, SEG_ROWS // 16, compact, 0)
            pltpu.sync_copy(
                seg_v,
                out_hbm.at[pl.ds(wid * (NSEG * SEG_LINES) + s * SEG_LINES,
                                 SEG_LINES)],
            )

    return _sc_gather


BB = 512
QB = BB // 4


def _tc_assemble(g_ref, xt_ref, wt_ref, b_ref, out_ref):
    for f in range(F):
        q = g_ref[f]
        q4 = q.reshape(QB, 4, D)
        out_ref[pl.ds(f * D, D), :] = q4.transpose(2, 0, 1).reshape(D, BB)
    acc = jax.lax.dot_general(
        wt_ref[...], xt_ref[...],
        (((0,), (0,)), ((), ())),
        preferred_element_type=jnp.float32,
    )
    out_ref[pl.ds(F * D, DD * D), :] = acc + b_ref[...]


def kernel(sparse_inputs, dense_inputs, table, W, b):
    tabler = table.reshape(250000, CHUNK)
    idx = sparse_inputs.T.astype(jnp.int32)
    idxq = (idx >> 2).reshape(NW, NCHUNK, CHUNK)
    idxr = ((idx & 3) * D).reshape(NW, IDX_PER_W)
    g = _get_sc_gather()(tabler, idxq, idxr)
    g3 = g.reshape(F, B // 4, CHUNK)
    xt = dense_inputs.T
    wt = W.T
    out_t = pl.pallas_call(
        _tc_assemble,
        grid=(B // BB,),
        in_specs=[
            pl.BlockSpec((F, QB, CHUNK), lambda i: (0, i, 0)),
            pl.BlockSpec((DD, BB), lambda i: (0, i)),
            pl.BlockSpec((DD, D * DD), lambda i: (0, 0)),
            pl.BlockSpec((D * DD, 1), lambda i: (0, 0)),
        ],
        out_specs=pl.BlockSpec((NOUT, BB), lambda i: (0, i)),
        out_shape=jax.ShapeDtypeStruct((NOUT, B), jnp.float32),
    )(g3, xt, wt, b.reshape(D * DD, 1))
    return out_t.reshape(F + DD, D, B).transpose(2, 0, 1)
